# baseline (speedup 1.0000x reference)
import functools

import jax
import jax.numpy as jnp
from jax import lax
from jax.experimental import pallas as pl
from jax.experimental.pallas import tpu as pltpu
from jax.experimental.pallas import tpu_sc as plsc

N = 10000
E = 320000
D = 128
H = 16

NCORES = 2
NSUB = 16
NW = NCORES * NSUB
CH = 80
EPT = E // NW
NCHUNK = EPT // CH
NZ = 624

ROW_BLK = 1000


_mesh = plsc.VectorSubcoreMesh(core_axis_name="c", subcore_axis_name="s")


@functools.partial(
    pl.kernel,
    out_type=jax.ShapeDtypeStruct((NCORES, N, D), jnp.float32),
    mesh=_mesh,
    scratch_types=[
        pltpu.VMEM((EPT,), jnp.int32),
        pltpu.VMEM((NCHUNK, CH), jnp.int32),
        pltpu.VMEM((CH, D), jnp.float32),
        pltpu.VMEM((CH, D), jnp.float32),
        pltpu.VMEM_SHARED((N, D), jnp.float32),
        pltpu.SemaphoreType.DMA,
        pltpu.SemaphoreType.DMA,
    ],
)
def _edge_aggr(m_hbm, ei_hbm, dst_hbm, zeros_hbm, out_hbm,
               sidx_v, didx_v, buf0, buf1, aggr_sh, sem0, sem1):
    c = lax.axis_index("c")
    s = lax.axis_index("s")
    wid = c * NSUB + s

    pltpu.sync_copy(ei_hbm.at[pl.ds(wid * EPT, EPT)], sidx_v)
    pltpu.sync_copy(dst_hbm.at[wid], didx_v)
    pltpu.sync_copy(zeros_hbm.at[pl.ds(0, NZ)], aggr_sh.at[pl.ds(s * NZ, NZ)])

    @pl.when(s == NSUB - 1)
    def _():
        pltpu.sync_copy(zeros_hbm.at[pl.ds(NZ, N - NSUB * NZ)],
                        aggr_sh.at[pl.ds(NSUB * NZ, N - NSUB * NZ)])

    plsc.subcore_barrier()

    bufs = (buf0, buf1)
    sems = (sem0, sem1)
    pltpu.async_copy(m_hbm.at[sidx_v.at[pl.ds(0, CH)]], buf0, sem0)
    pltpu.async_copy(m_hbm.at[sidx_v.at[pl.ds(CH, CH)]], buf1, sem1)

    def body(t2, carry):
        for j in range(2):
            tt = t2 * 2 + j
            pltpu.make_async_copy(m_hbm.at[sidx_v.at[pl.ds(tt * CH, CH)]],
                                  bufs[j], sems[j]).wait()
            pltpu.sync_copy(bufs[j], aggr_sh.at[didx_v.at[tt]], add=True)
            nxt = tt + 2

            @pl.when(nxt < NCHUNK)
            def _():
                pltpu.async_copy(m_hbm.at[sidx_v.at[pl.ds(nxt * CH, CH)]],
                                 bufs[j], sems[j])
        return carry

    lax.fori_loop(0, NCHUNK // 2, body, 0)

    last = NCHUNK - 1
    pltpu.make_async_copy(m_hbm.at[sidx_v.at[pl.ds(last * CH, CH)]],
                          buf0, sem0).wait()
    pltpu.sync_copy(buf0, aggr_sh.at[didx_v.at[last]], add=True)

    plsc.subcore_barrier()
    pltpu.sync_copy(aggr_sh.at[pl.ds(s * NZ, NZ)],
                    out_hbm.at[c, pl.ds(s * NZ, NZ)])

    @pl.when(s == NSUB - 1)
    def _():
        pltpu.sync_copy(aggr_sh.at[pl.ds(NSUB * NZ, N - NSUB * NZ)],
                        out_hbm.at[c, pl.ds(NSUB * NZ, N - NSUB * NZ)])


def _msg_body(x_ref, w1, b1, w2, b2, o_ref):
    h = jnp.maximum(
        jnp.dot(x_ref[...], w1[...], preferred_element_type=jnp.float32)
        + b1[...], 0.0)
    m = jnp.maximum(
        jnp.dot(h, w2[...], preferred_element_type=jnp.float32) + b2[...], 0.0)
    o_ref[...] = m.astype(o_ref.dtype)


def _full(shape):
    return pl.BlockSpec(shape, lambda i: (0, 0))


def _node_msg(x, w1t, b1, w2t, b2):
    return pl.pallas_call(
        _msg_body,
        grid=(N // ROW_BLK,),
        in_specs=[
            pl.BlockSpec((ROW_BLK, D), lambda i: (i, 0)),
            _full((D, H)), _full((1, H)), _full((H, D)), _full((1, D)),
        ],
        out_specs=pl.BlockSpec((ROW_BLK, D), lambda i: (i, 0)),
        out_shape=jax.ShapeDtypeStruct((N, D), jnp.float32),
    )(x, w1t, b1, w2t, b2)


def _upd_body(p_ref, x_ref, u1a, u1b, ub1, u2t, ub2, o_ref):
    aggr = p_ref[0] + p_ref[1]
    h = jnp.maximum(
        jnp.dot(aggr, u1a[...], preferred_element_type=jnp.float32)
        + jnp.dot(x_ref[...], u1b[...], preferred_element_type=jnp.float32)
        + ub1[...], 0.0)
    o_ref[...] = jnp.maximum(
        jnp.dot(h, u2t[...], preferred_element_type=jnp.float32) + ub2[...],
        0.0)


def _upd_msg_body(p_ref, x_ref, u1a, u1b, ub1, u2t, ub2,
                  m1t, mb1, m2t, mb2, nx_ref, m_ref):
    _upd_body(p_ref, x_ref, u1a, u1b, ub1, u2t, ub2, nx_ref)
    _msg_body(nx_ref, m1t, mb1, m2t, mb2, m_ref)


def _pblk():
    return pl.BlockSpec((NCORES, ROW_BLK, D), lambda i: (0, i, 0))


def _update(partials, x, u1a, u1b, ub1, u2t, ub2):
    blk = pl.BlockSpec((ROW_BLK, D), lambda i: (i, 0))
    return pl.pallas_call(
        _upd_body,
        grid=(N // ROW_BLK,),
        in_specs=[_pblk(), blk,
                  _full((D, H)), _full((D, H)), _full((1, H)),
                  _full((H, D)), _full((1, D))],
        out_specs=blk,
        out_shape=jax.ShapeDtypeStruct((N, D), jnp.float32),
    )(partials, x, u1a, u1b, ub1, u2t, ub2)


def _update_msg(partials, x, u1a, u1b, ub1, u2t, ub2, m1t, mb1, m2t, mb2):
    blk = pl.BlockSpec((ROW_BLK, D), lambda i: (i, 0))
    return pl.pallas_call(
        _upd_msg_body,
        grid=(N // ROW_BLK,),
        in_specs=[_pblk(), blk,
                  _full((D, H)), _full((D, H)), _full((1, H)),
                  _full((H, D)), _full((1, D)),
                  _full((D, H)), _full((1, H)), _full((H, D)), _full((1, D))],
        out_specs=[blk, blk],
        out_shape=[jax.ShapeDtypeStruct((N, D), jnp.float32),
                   jax.ShapeDtypeStruct((N, D), jnp.float32)],
    )(partials, x, u1a, u1b, ub1, u2t, ub2, m1t, mb1, m2t, mb2)


def _prep_mlp(p):
    return (p['W1'].T, p['b1'].reshape(1, -1), p['W2'].T,
            p['b2'].reshape(1, -1))


def kernel(x, edge_index, params):
    ei = edge_index.astype(jnp.int32)
    src = ei[0]
    dst = ei[1].reshape(NW, NCHUNK, CH)
    zeros = jnp.zeros((NZ + N - NSUB * NZ, D), jnp.float32)

    msg_w = [_prep_mlp(p['mlp']) for p in params]
    upd_w = []
    for p in params:
        u1t = p['update']['W1'].T
        upd_w.append((u1t[:D], u1t[D:], p['update']['b1'].reshape(1, -1),
                      p['update']['W2'].T, p['update']['b2'].reshape(1, -1)))

    m = _node_msg(x, *msg_w[0])
    for l in range(3):
        partials = _edge_aggr(m, src, dst, zeros)
        if l < 2:
            x, m = _update_msg(partials, x, *upd_w[l], *msg_w[l + 1])
        else:
            x = _update(partials, x, *upd_w[l])
    return x


# speedup vs baseline: 1.0171x; 1.0171x over previous
import functools

import jax
import jax.numpy as jnp
from jax import lax
from jax.experimental import pallas as pl
from jax.experimental.pallas import tpu as pltpu
from jax.experimental.pallas import tpu_sc as plsc

N = 10000
E = 320000
D = 128
H = 16

NCORES = 2
NSUB = 16
NW = NCORES * NSUB
CH = 80
EPT = E // NW
NCHUNK = EPT // CH
NZ = 624

ROW_BLK = 2000


_mesh = plsc.VectorSubcoreMesh(core_axis_name="c", subcore_axis_name="s")


@functools.partial(
    pl.kernel,
    out_type=jax.ShapeDtypeStruct((NCORES, N, D), jnp.float32),
    mesh=_mesh,
    scratch_types=[
        pltpu.VMEM((EPT,), jnp.int32),
        pltpu.VMEM((NCHUNK, CH), jnp.int32),
        pltpu.VMEM((CH, D), jnp.float32),
        pltpu.VMEM((CH, D), jnp.float32),
        pltpu.VMEM_SHARED((N, D), jnp.float32),
        pltpu.SemaphoreType.DMA,
        pltpu.SemaphoreType.DMA,
    ],
)
def _edge_aggr(m_hbm, ei_hbm, dst_hbm, zeros_hbm, out_hbm,
               sidx_v, didx_v, buf0, buf1, aggr_sh, sem0, sem1):
    c = lax.axis_index("c")
    s = lax.axis_index("s")
    wid = c * NSUB + s

    pltpu.sync_copy(ei_hbm.at[pl.ds(wid * EPT, EPT)], sidx_v)
    pltpu.sync_copy(dst_hbm.at[wid], didx_v)
    pltpu.sync_copy(zeros_hbm.at[pl.ds(0, NZ)], aggr_sh.at[pl.ds(s * NZ, NZ)])

    @pl.when(s == NSUB - 1)
    def _():
        pltpu.sync_copy(zeros_hbm.at[pl.ds(NZ, N - NSUB * NZ)],
                        aggr_sh.at[pl.ds(NSUB * NZ, N - NSUB * NZ)])

    plsc.subcore_barrier()

    bufs = (buf0, buf1)
    sems = (sem0, sem1)
    pltpu.async_copy(m_hbm.at[sidx_v.at[pl.ds(0, CH)]], buf0, sem0)
    pltpu.async_copy(m_hbm.at[sidx_v.at[pl.ds(CH, CH)]], buf1, sem1)

    def body(t2, carry):
        for j in range(2):
            tt = t2 * 2 + j
            pltpu.make_async_copy(m_hbm.at[sidx_v.at[pl.ds(tt * CH, CH)]],
                                  bufs[j], sems[j]).wait()
            pltpu.sync_copy(bufs[j], aggr_sh.at[didx_v.at[tt]], add=True)
            nxt = tt + 2

            @pl.when(nxt < NCHUNK)
            def _():
                pltpu.async_copy(m_hbm.at[sidx_v.at[pl.ds(nxt * CH, CH)]],
                                 bufs[j], sems[j])
        return carry

    lax.fori_loop(0, NCHUNK // 2, body, 0)

    last = NCHUNK - 1
    pltpu.make_async_copy(m_hbm.at[sidx_v.at[pl.ds(last * CH, CH)]],
                          buf0, sem0).wait()
    pltpu.sync_copy(buf0, aggr_sh.at[didx_v.at[last]], add=True)

    plsc.subcore_barrier()
    pltpu.sync_copy(aggr_sh.at[pl.ds(s * NZ, NZ)],
                    out_hbm.at[c, pl.ds(s * NZ, NZ)])

    @pl.when(s == NSUB - 1)
    def _():
        pltpu.sync_copy(aggr_sh.at[pl.ds(NSUB * NZ, N - NSUB * NZ)],
                        out_hbm.at[c, pl.ds(NSUB * NZ, N - NSUB * NZ)])


def _msg_body(x_ref, w1, b1, w2, b2, o_ref):
    h = jnp.maximum(
        jnp.dot(x_ref[...], w1[...], preferred_element_type=jnp.float32)
        + b1[...], 0.0)
    m = jnp.maximum(
        jnp.dot(h, w2[...], preferred_element_type=jnp.float32) + b2[...], 0.0)
    o_ref[...] = m.astype(o_ref.dtype)


def _full(shape):
    return pl.BlockSpec(shape, lambda i: (0, 0))


def _node_msg(x, w1t, b1, w2t, b2):
    return pl.pallas_call(
        _msg_body,
        grid=(N // ROW_BLK,),
        in_specs=[
            pl.BlockSpec((ROW_BLK, D), lambda i: (i, 0)),
            _full((D, H)), _full((1, H)), _full((H, D)), _full((1, D)),
        ],
        out_specs=pl.BlockSpec((ROW_BLK, D), lambda i: (i, 0)),
        out_shape=jax.ShapeDtypeStruct((N, D), jnp.float32),
    )(x, w1t, b1, w2t, b2)


def _upd_body(p_ref, x_ref, u1a, u1b, ub1, u2t, ub2, o_ref):
    aggr = p_ref[0] + p_ref[1]
    h = jnp.maximum(
        jnp.dot(aggr, u1a[...], preferred_element_type=jnp.float32)
        + jnp.dot(x_ref[...], u1b[...], preferred_element_type=jnp.float32)
        + ub1[...], 0.0)
    o_ref[...] = jnp.maximum(
        jnp.dot(h, u2t[...], preferred_element_type=jnp.float32) + ub2[...],
        0.0)


def _upd_msg_body(p_ref, x_ref, u1a, u1b, ub1, u2t, ub2,
                  m1t, mb1, m2t, mb2, nx_ref, m_ref):
    _upd_body(p_ref, x_ref, u1a, u1b, ub1, u2t, ub2, nx_ref)
    _msg_body(nx_ref, m1t, mb1, m2t, mb2, m_ref)


def _pblk():
    return pl.BlockSpec((NCORES, ROW_BLK, D), lambda i: (0, i, 0))


def _update(partials, x, u1a, u1b, ub1, u2t, ub2):
    blk = pl.BlockSpec((ROW_BLK, D), lambda i: (i, 0))
    return pl.pallas_call(
        _upd_body,
        grid=(N // ROW_BLK,),
        in_specs=[_pblk(), blk,
                  _full((D, H)), _full((D, H)), _full((1, H)),
                  _full((H, D)), _full((1, D))],
        out_specs=blk,
        out_shape=jax.ShapeDtypeStruct((N, D), jnp.float32),
    )(partials, x, u1a, u1b, ub1, u2t, ub2)


def _update_msg(partials, x, u1a, u1b, ub1, u2t, ub2, m1t, mb1, m2t, mb2):
    blk = pl.BlockSpec((ROW_BLK, D), lambda i: (i, 0))
    return pl.pallas_call(
        _upd_msg_body,
        grid=(N // ROW_BLK,),
        in_specs=[_pblk(), blk,
                  _full((D, H)), _full((D, H)), _full((1, H)),
                  _full((H, D)), _full((1, D)),
                  _full((D, H)), _full((1, H)), _full((H, D)), _full((1, D))],
        out_specs=[blk, blk],
        out_shape=[jax.ShapeDtypeStruct((N, D), jnp.float32),
                   jax.ShapeDtypeStruct((N, D), jnp.float32)],
    )(partials, x, u1a, u1b, ub1, u2t, ub2, m1t, mb1, m2t, mb2)


def _prep_mlp(p):
    return (p['W1'].T, p['b1'].reshape(1, -1), p['W2'].T,
            p['b2'].reshape(1, -1))


def kernel(x, edge_index, params):
    ei = edge_index.astype(jnp.int32)
    src = ei.reshape(2 * E)
    dst = ei[1].reshape(NW, NCHUNK, CH)
    zeros = jnp.zeros((NZ + N - NSUB * NZ, D), jnp.float32)

    msg_w = [_prep_mlp(p['mlp']) for p in params]
    upd_w = []
    for p in params:
        u1t = p['update']['W1'].T
        upd_w.append((u1t[:D], u1t[D:], p['update']['b1'].reshape(1, -1),
                      p['update']['W2'].T, p['update']['b2'].reshape(1, -1)))

    m = _node_msg(x, *msg_w[0])
    for l in range(3):
        partials = _edge_aggr(m, src, dst, zeros)
        if l < 2:
            x, m = _update_msg(partials, x, *upd_w[l], *msg_w[l + 1])
        else:
            x = _update(partials, x, *upd_w[l])
    return x
